# Initial kernel scaffold; baseline (speedup 1.0000x reference)
import functools

import jax
import jax.numpy as jnp
from jax import lax
from jax.experimental import pallas as pl
from jax.experimental.pallas import tpu as pltpu
from jax.experimental.pallas import tpu_sc as plsc

NUM_TABLES = 26
VOCAB = 100000
D = 16
BATCH = 4096
L = 20

NC = 2
NS = 16
NW = NC * NS
B_PER_W = BATCH // NW
ROWS = B_PER_W * L
INV_L = 1.0 / L

_mesh = plsc.VectorSubcoreMesh(core_axis_name="c", subcore_axis_name="s")


@functools.partial(
    pl.kernel,
    mesh=_mesh,
    out_type=jax.ShapeDtypeStruct((BATCH, NUM_TABLES * D), jnp.float32),
    scratch_types=[
        pltpu.VMEM((ROWS,), jnp.int32),
        pltpu.VMEM((ROWS, D), jnp.float32),
        pltpu.VMEM((B_PER_W, NUM_TABLES * D), jnp.float32),
        pltpu.SemaphoreType.DMA,
    ],
)
def _ebag(idx_hbm, tab_hbm, out_hbm, idx_v, rows_v, ob_v, gsem):
    wid = lax.axis_index("s") * NC + lax.axis_index("c")
    b0 = wid * B_PER_W

    def table_step(t, _):
        ofs = t * (BATCH * L) + b0 * L
        pltpu.sync_copy(idx_hbm.at[pl.ds(ofs, ROWS)], idx_v)
        pltpu.async_copy(tab_hbm.at[idx_v], rows_v, gsem).wait()

        d0 = t * D

        def bag_step(i, _):
            base = i * L
            acc = rows_v[base, :]
            for l in range(1, L):
                acc = acc + rows_v[base + l, :]
            ob_v[i, pl.ds(d0, D)] = acc * INV_L
            return 0

        lax.fori_loop(0, B_PER_W, bag_step, 0)
        return 0

    lax.fori_loop(0, NUM_TABLES, table_step, 0)
    pltpu.sync_copy(ob_v, out_hbm.at[pl.ds(b0, B_PER_W)])


def kernel(indices, offsets, tables):
    del offsets
    flat = tables.reshape(NUM_TABLES * VOCAB, D)
    return _ebag(indices, flat)


# speedup vs baseline: 171.0828x; 171.0828x over previous
import functools

import jax
import jax.numpy as jnp
from jax import lax
from jax.experimental import pallas as pl
from jax.experimental.pallas import tpu as pltpu
from jax.experimental.pallas import tpu_sc as plsc

NUM_TABLES = 26
VOCAB = 100000
D = 16
BATCH = 4096
L = 20

NC = 2
NS = 16
NW = NC * NS
B_PER_W = BATCH // NW
ROWS = B_PER_W * L
INV_L = 1.0 / L

_mesh = plsc.VectorSubcoreMesh(core_axis_name="c", subcore_axis_name="s")


@functools.partial(
    pl.kernel,
    mesh=_mesh,
    compiler_params=pltpu.CompilerParams(use_tc_tiling_on_sc=False),
    out_type=jax.ShapeDtypeStruct((BATCH, NUM_TABLES * D), jnp.float32),
    scratch_types=[
        pltpu.VMEM((ROWS,), jnp.int32),
        pltpu.VMEM((ROWS, D), jnp.float32),
        pltpu.VMEM((B_PER_W, NUM_TABLES * D), jnp.float32),
        pltpu.SemaphoreType.DMA,
    ],
)
def _ebag(idx_hbm, tab_hbm, out_hbm, idx_v, rows_v, ob_v, gsem):
    wid = lax.axis_index("s") * NC + lax.axis_index("c")
    b0 = wid * B_PER_W

    def table_step(t, _):
        ofs = t * (BATCH * L) + b0 * L
        pltpu.sync_copy(idx_hbm.at[pl.ds(ofs, ROWS)], idx_v)
        pltpu.async_copy(tab_hbm.at[idx_v], rows_v, gsem).wait()

        d0 = t * D

        def bag_step(i, _):
            base = i * L
            acc = rows_v[base, :]
            for l in range(1, L):
                acc = acc + rows_v[base + l, :]
            ob_v[i, pl.ds(d0, D)] = acc * INV_L
            return 0

        lax.fori_loop(0, B_PER_W, bag_step, 0)
        return 0

    lax.fori_loop(0, NUM_TABLES, table_step, 0)
    pltpu.sync_copy(ob_v, out_hbm.at[pl.ds(b0, B_PER_W)])


def kernel(indices, offsets, tables):
    del offsets
    flat = tables.reshape(NUM_TABLES * VOCAB, D)
    return _ebag(indices, flat)
